# Initial kernel scaffold; baseline (speedup 1.0000x reference)
import jax
import jax.numpy as jnp
from jax.experimental import pallas as pl


def kernel(x, edge_index, edge_weight, bw0, sw0, sc0, bw1, sw1, sc1, bw2, sw2, sc2, bw3, sw3, sc3, bw4, sw4, sc4, bw5, sw5, sc5):
    raise NotImplementedError("write your pallas kernel here")


# speedup vs baseline: 3.9134x; 3.9134x over previous
import functools

import jax
import jax.numpy as jnp
import numpy as np
from jax import lax
from jax.experimental import pallas as pl
from jax.experimental.pallas import tpu as pltpu
from jax.experimental.pallas import tpu_sc as plsc

N = 10000
E = 320000
D = 128
G = 5
K = 3
NB = G + K

NC = 2
NS = 16
NW = NC * NS
EPW = E // NW
CHUNK = 80
NCHUNK = EPW // CHUNK
NPAD = 10240
RPT = NPAD // NS

_H = 2.0 / G
_GRID = (np.arange(-K, G + K + 1, dtype=np.float32) * np.float32(_H)
         - np.float32(1.0))


def _spmm_sc(h, src, dst, w, zeros):
    mesh = plsc.VectorSubcoreMesh(core_axis_name="c", subcore_axis_name="s")

    @functools.partial(
        pl.kernel,
        out_type=jax.ShapeDtypeStruct((NC, NPAD, D), jnp.float32),
        mesh=mesh,
        scratch_types=[
            pltpu.VMEM((CHUNK,), jnp.int32),
            pltpu.VMEM((CHUNK,), jnp.int32),
            pltpu.VMEM((CHUNK,), jnp.float32),
            pltpu.VMEM((CHUNK, D), jnp.float32),
            pltpu.VMEM_SHARED((NPAD, D), jnp.float32),
            pltpu.SemaphoreType.DMA,
        ],
    )
    def k(h_hbm, src_hbm, dst_hbm, w_hbm, z_hbm, out_hbm,
          src_v, dst_v, w_v, rows_v, acc_sh, sem):
        c = lax.axis_index("c")
        s = lax.axis_index("s")
        wid = c * NS + s
        pltpu.sync_copy(z_hbm.at[pl.ds(s * RPT, RPT)],
                        acc_sh.at[pl.ds(s * RPT, RPT)])
        plsc.subcore_barrier()
        base = wid * EPW

        def body(kk, carry):
            off = base + kk * CHUNK
            pltpu.sync_copy(src_hbm.at[pl.ds(off, CHUNK)], src_v)
            pltpu.sync_copy(dst_hbm.at[pl.ds(off, CHUNK)], dst_v)
            pltpu.sync_copy(w_hbm.at[pl.ds(off, CHUNK)], w_v)
            pltpu.async_copy(h_hbm.at[src_v], rows_v, sem).wait()

            for g in range(CHUNK // 16):
                w16 = w_v[pl.ds(g * 16, 16)]
                for i in range(16):
                    wvec = jnp.full((16,), w16[i], dtype=jnp.float32)
                    e = g * 16 + i
                    for j in range(D // 16):
                        sl = pl.ds(j * 16, 16)
                        rows_v[e, sl] = rows_v[e, sl] * wvec
            pltpu.sync_copy(rows_v, acc_sh.at[dst_v], add=True)
            return carry

        lax.fori_loop(0, NCHUNK, body, 0)
        plsc.subcore_barrier()
        pltpu.sync_copy(acc_sh.at[pl.ds(s * RPT, RPT)],
                        out_hbm.at[c, pl.ds(s * RPT, RPT)])

    return k(h, src, dst, w, zeros)


def _bases(x):
    g = [float(v) for v in _GRID]
    bs = [jnp.logical_and(x >= g[j], x < g[j + 1]).astype(jnp.float32)
          for j in range(G + 2 * K)]
    for k in range(1, K + 1):
        nxt = []
        for j in range(len(bs) - 1):
            d1 = float(np.float32(g[j + k]) - np.float32(g[j]))
            d2 = float(np.float32(g[j + k + 1]) - np.float32(g[j + 1]))
            nxt.append((x - g[j]) / d1 * bs[j] + (g[j + k + 1] - x) / d2 * bs[j + 1])
        bs = nxt
    return bs


def _kan_apply(x, wref):
    sil = x * jax.nn.sigmoid(x)
    feats = jnp.concatenate([sil] + _bases(x), axis=1)
    return jnp.dot(feats, wref, preferred_element_type=jnp.float32)


def _kan_pair_tc(parts, wa, wb, bt=400):
    def body(p_ref, wa_ref, wb_ref, out_ref):
        x = p_ref[0] + p_ref[1]
        y = _kan_apply(x, wa_ref[...])
        out_ref[...] = _kan_apply(y, wb_ref[...])

    grid = (N // bt,)
    wdim = D * (NB + 1)
    return pl.pallas_call(
        body,
        grid=grid,
        in_specs=[
            pl.BlockSpec((NC, bt, D), lambda i: (0, i, 0)),
            pl.BlockSpec((wdim, D), lambda i: (0, 0)),
            pl.BlockSpec((wdim, D), lambda i: (0, 0)),
        ],
        out_specs=pl.BlockSpec((bt, D), lambda i: (i, 0)),
        out_shape=jax.ShapeDtypeStruct((N, D), jnp.float32),
    )(parts, wa, wb)


def _fold_weights(bw, sw, sc):
    ssw = sw * sc[:, :, None]
    wsp = ssw.transpose(2, 1, 0).reshape(NB * D, D)
    return jnp.concatenate([bw.T, wsp], axis=0)


def kernel(x, edge_index, edge_weight,
           bw0, sw0, sc0, bw1, sw1, sc1, bw2, sw2, sc2,
           bw3, sw3, sc3, bw4, sw4, sc4, bw5, sw5, sc5):
    dst = edge_index[0]
    src = edge_index[1]
    zeros = jnp.zeros((NPAD, D), jnp.float32)
    ws = [_fold_weights(bw0, sw0, sc0), _fold_weights(bw1, sw1, sc1),
          _fold_weights(bw2, sw2, sc2), _fold_weights(bw3, sw3, sc3),
          _fold_weights(bw4, sw4, sc4), _fold_weights(bw5, sw5, sc5)]
    h = x
    for i in range(3):
        parts = _spmm_sc(h, src, dst, edge_weight, zeros)
        h = _kan_pair_tc(parts, ws[2 * i], ws[2 * i + 1])
    return h
